# baseline (speedup 1.0000x reference)
import jax
import jax.numpy as jnp
from jax import lax
from jax.experimental import pallas as pl
from jax.experimental.pallas import tpu as pltpu
from jax.experimental.pallas import tpu_sc as plsc

SITU_DIM = 100000
S = 4
D = 128
B = 4096
L = 50
LP = 56

NC, NS = 2, 16
NW = NC * NS
BPW = B // NW
NBC = 2
NCHUNK = BPW // NBC
NBUF = 4


def _sc_body(table_hbm, idxr_hbm, o0, o1, o2, o3, idx_v, buf, *sems):
    gsem = sems[:NBUF]
    ssem = sems[NBUF:]
    outs = (o0, o1, o2, o3)
    wid = lax.axis_index("s") * NC + lax.axis_index("c")

    pltpu.sync_copy(idxr_hbm.at[wid], idx_v)

    def g_start(i, j, b):
        for t in range(NBC):
            pltpu.make_async_copy(
                table_hbm.at[idx_v.at[i, j * NBC + t, pl.ds(0, L)]],
                buf.at[b, t], gsem[b]).start()

    def g_wait(b):
        for t in range(NBC):
            pltpu.make_async_copy(
                table_hbm.at[idx_v.at[0, 0, pl.ds(0, L)]],
                buf.at[b, t], gsem[b]).wait()

    def s_start(i, j, b):
        pltpu.make_async_copy(
            buf.at[b], outs[i].at[pl.ds(wid * BPW + j * NBC, NBC)], ssem[b]
        ).start()

    def s_wait(i, b):
        pltpu.make_async_copy(
            buf.at[b], outs[i].at[pl.ds(0, NBC)], ssem[b]).wait()

    for i in range(S):
        g_start(i, 0, 0)
        g_start(i, 1, 1)
        for j in range(2):
            g_wait(j)
            s_start(i, j, j)
            g_start(i, j + 2, j + 2)

        @pl.loop(2, 62, step=NBUF)
        def _(g):
            for k in range(NBUF):
                b = (2 + k) % NBUF
                j = g + k
                g_wait(b)
                s_start(i, j, b)
                s_wait(i, (b + 2) % NBUF)
                g_start(i, j + 2, (b + 2) % NBUF)

        for j in range(62, 64):
            b = j % NBUF
            g_wait(b)
            s_start(i, j, b)
            s_wait(i, (j - 2) % NBUF)
        s_wait(i, 62 % NBUF)
        s_wait(i, 63 % NBUF)


@jax.jit
def _run(table, idxr):
    out_sds = tuple(
        jax.ShapeDtypeStruct((B, L, D), jnp.float32) for _ in range(S))
    mesh = plsc.VectorSubcoreMesh(core_axis_name="c", subcore_axis_name="s")
    f = pl.kernel(
        _sc_body,
        out_type=out_sds,
        mesh=mesh,
        compiler_params=pltpu.CompilerParams(use_tc_tiling_on_sc=True),
        scratch_types=[
            pltpu.VMEM((S, BPW, LP), jnp.int32),
            pltpu.VMEM((NBUF, NBC, L, D), jnp.float32),
        ] + [pltpu.SemaphoreType.DMA] * (2 * NBUF),
    )
    return f(table, idxr)


def kernel(history_context_features, emb_weight):
    idx_t = jnp.transpose(history_context_features, (2, 0, 1))
    idx_p = jnp.pad(idx_t, ((0, 0), (0, 0), (0, LP - L)))
    idxr = jnp.transpose(idx_p.reshape(S, NW, BPW, LP), (1, 0, 2, 3))
    return _run(emb_weight, idxr)


# speedup vs baseline: 1.0158x; 1.0158x over previous
import jax
import jax.numpy as jnp
from jax import lax
from jax.experimental import pallas as pl
from jax.experimental.pallas import tpu as pltpu
from jax.experimental.pallas import tpu_sc as plsc

SITU_DIM = 100000
S = 4
D = 128
B = 4096
L = 50
LP = 56

NC, NS = 2, 16
NW = NC * NS
BPW = B // NW
NBC = 2
NCHUNK = BPW // NBC
NBUF = 4


def _sc_body(table_hbm, idxr_hbm, out, idx_v, buf, *sems):
    gsem = sems[:NBUF]
    ssem = sems[NBUF:]
    wid = lax.axis_index("s") * NC + lax.axis_index("c")

    pltpu.sync_copy(idxr_hbm.at[wid], idx_v)

    def g_start(j, b):
        for t in range(NBC):
            pltpu.make_async_copy(
                table_hbm.at[idx_v.at[j * NBC + t, pl.ds(0, L)]],
                buf.at[b, t], gsem[b]).start()

    def g_wait(b):
        for t in range(NBC):
            pltpu.make_async_copy(
                table_hbm.at[idx_v.at[0, pl.ds(0, L)]],
                buf.at[b, t], gsem[b]).wait()

    def s_start(j, b):
        pltpu.make_async_copy(
            buf.at[b], out.at[pl.ds(wid * BPW + j * NBC, NBC)], ssem[b]
        ).start()

    def s_wait(b):
        pltpu.make_async_copy(
            buf.at[b], out.at[pl.ds(0, NBC)], ssem[b]).wait()

    g_start(0, 0)
    g_start(1, 1)
    for j in range(2):
        g_wait(j)
        s_start(j, j)
        g_start(j + 2, j + 2)

    @pl.loop(2, 62, step=NBUF)
    def _(g):
        for k in range(NBUF):
            b = (2 + k) % NBUF
            j = g + k
            g_wait(b)
            s_start(j, b)
            s_wait((b + 2) % NBUF)
            g_start(j + 2, (b + 2) % NBUF)

    for j in range(62, 64):
        b = j % NBUF
        g_wait(b)
        s_start(j, b)
        s_wait((j - 2) % NBUF)
    s_wait(62 % NBUF)
    s_wait(63 % NBUF)


@jax.jit
def _run(table, idxr):
    mesh = plsc.VectorSubcoreMesh(core_axis_name="c", subcore_axis_name="s")
    outs = []
    for i in range(S):
        f = pl.kernel(
            _sc_body,
            out_type=jax.ShapeDtypeStruct((B, L, D), jnp.float32),
            mesh=mesh,
            scratch_types=[
                pltpu.VMEM((BPW, LP), jnp.int32),
                pltpu.VMEM((NBUF, NBC, L, D), jnp.float32),
            ] + [pltpu.SemaphoreType.DMA] * (2 * NBUF),
            name=f"emb_gather_{i}",
        )
        outs.append(f(table, idxr[:, i]))
    return tuple(outs)


def kernel(history_context_features, emb_weight):
    idx_t = jnp.transpose(history_context_features, (2, 0, 1))
    idx_p = jnp.pad(idx_t, ((0, 0), (0, 0), (0, LP - L)))
    idxr = jnp.transpose(idx_p.reshape(S, NW, BPW, LP), (1, 0, 2, 3))
    return _run(emb_weight, idxr)


# speedup vs baseline: 1.9380x; 1.9079x over previous
import jax
import jax.numpy as jnp
from jax import lax
from jax.experimental import pallas as pl
from jax.experimental.pallas import tpu as pltpu
from jax.experimental.pallas import tpu_sc as plsc

SITU_DIM = 100000
S = 4
D = 128
B = 4096
L = 50

NC, NS = 2, 16
NW = NC * NS
C = B // NW
NBUF = 5


def _sc_body(table_hbm, idxr_hbm, o0, o1, o2, o3, idx_v, buf, *sems):
    gsem = sems[:NBUF]
    ssem = sems[NBUF:]
    outs = (o0, o1, o2, o3)
    wid = lax.axis_index("s") * NC + lax.axis_index("c")

    pltpu.sync_copy(idxr_hbm.at[wid], idx_v)

    def g_start(i, l, b):
        pltpu.make_async_copy(
            table_hbm.at[idx_v.at[i, l]], buf.at[b], gsem[b]).start()

    def g_wait(b):
        pltpu.make_async_copy(
            table_hbm.at[idx_v.at[0, 0]], buf.at[b], gsem[b]).wait()

    def s_start(i, l, b):
        pltpu.make_async_copy(
            buf.at[b], outs[i].at[l, pl.ds(wid * C, C)], ssem[b]).start()

    def s_wait(i, b):
        pltpu.make_async_copy(
            buf.at[b], outs[i].at[0, pl.ds(0, C)], ssem[b]).wait()

    for i in range(S):
        for l in range(3):
            g_start(i, l, l)
        for l in range(2):
            g_wait(l)
            s_start(i, l, l)
            g_start(i, l + 3, l + 3)

        @pl.loop(2, 47, step=NBUF)
        def _(g):
            for k in range(NBUF):
                b = (2 + k) % NBUF
                l = g + k
                g_wait(b)
                s_start(i, l, b)
                s_wait(i, (b + 3) % NBUF)
                g_start(i, l + 3, (b + 3) % NBUF)

        for l in range(47, 50):
            b = l % NBUF
            g_wait(b)
            s_start(i, l, b)
            s_wait(i, (l - 2) % NBUF)
        s_wait(i, 48 % NBUF)
        s_wait(i, 49 % NBUF)


@jax.jit
def _run(table, idxr):
    out_sds = tuple(
        jax.ShapeDtypeStruct((L, B, D), jnp.float32) for _ in range(S))
    mesh = plsc.VectorSubcoreMesh(core_axis_name="c", subcore_axis_name="s")
    f = pl.kernel(
        _sc_body,
        out_type=out_sds,
        mesh=mesh,
        scratch_types=[
            pltpu.VMEM((S, L, C), jnp.int32),
            pltpu.VMEM((NBUF, C, D), jnp.float32),
        ] + [pltpu.SemaphoreType.DMA] * (2 * NBUF),
        name="emb_gather",
    )
    outs = f(table, idxr)
    return tuple(jnp.transpose(o, (1, 0, 2)) for o in outs)


def kernel(history_context_features, emb_weight):
    idx_t = jnp.transpose(history_context_features, (2, 1, 0))
    idxr = jnp.transpose(idx_t.reshape(S, L, NW, C), (2, 0, 1, 3))
    return _run(emb_weight, idxr)


# speedup vs baseline: 1.9464x; 1.0043x over previous
import jax
import jax.numpy as jnp
from jax import lax
from jax.experimental import pallas as pl
from jax.experimental.pallas import tpu as pltpu
from jax.experimental.pallas import tpu_sc as plsc

SITU_DIM = 100000
S = 4
D = 128
B = 4096
L = 50

NC, NS = 2, 16
NW = NC * NS
C = B // NW
NBUF = 6
PF = 4
NQ = S * L


def _sc_body(table_hbm, idxr_hbm, o0, o1, o2, o3, idx_v, buf, *sems):
    gsem = sems[:NBUF]
    ssem = sems[NBUF:]
    outs = (o0, o1, o2, o3)
    wid = lax.axis_index("s") * NC + lax.axis_index("c")

    pltpu.sync_copy(idxr_hbm.at[wid], idx_v)

    def g_start(i, l, b):
        pltpu.make_async_copy(
            table_hbm.at[idx_v.at[i, l]], buf.at[b], gsem[b]).start()

    def g_wait(b):
        pltpu.make_async_copy(
            table_hbm.at[idx_v.at[0, 0]], buf.at[b], gsem[b]).wait()

    def s_start(i, l, b):
        pltpu.make_async_copy(
            buf.at[b], outs[i].at[l, pl.ds(wid * C, C)], ssem[b]).start()

    def s_wait(b):
        pltpu.make_async_copy(
            buf.at[b], o0.at[0, pl.ds(0, C)], ssem[b]).wait()

    def step_static(q):
        b = q % NBUF
        g_wait(b)
        s_start(q // L, q % L, b)
        if q >= NBUF - PF:
            s_wait((q - (NBUF - PF)) % NBUF)
        if q + PF < NQ:
            g_start((q + PF) // L, (q + PF) % L, (q + PF) % NBUF)

    for q in range(PF):
        g_start(q // L, q % L, q % NBUF)

    step_static(0)
    step_static(1)
    for i in range(S):
        q0 = i * L + (2 if i == 0 else 0)

        @pl.loop(q0, q0 + 42, step=NBUF)
        def _(g):
            for k in range(NBUF):
                b = (q0 + k) % NBUF
                q = g + k
                g_wait(b)
                s_start(i, q - i * L, b)
                s_wait((b + PF) % NBUF)
                g_start(i, q - i * L + PF, (b + PF) % NBUF)

        for q in range(i * L + (44 if i == 0 else 42), (i + 1) * L):
            step_static(q)

    s_wait((NQ - 2) % NBUF)
    s_wait((NQ - 1) % NBUF)


@jax.jit
def _run(table, idxr):
    out_sds = tuple(
        jax.ShapeDtypeStruct((L, B, D), jnp.float32) for _ in range(S))
    mesh = plsc.VectorSubcoreMesh(core_axis_name="c", subcore_axis_name="s")
    f = pl.kernel(
        _sc_body,
        out_type=out_sds,
        mesh=mesh,
        scratch_types=[
            pltpu.VMEM((S, L, C), jnp.int32),
            pltpu.VMEM((NBUF, C, D), jnp.float32),
        ] + [pltpu.SemaphoreType.DMA] * (2 * NBUF),
        name="emb_gather",
    )
    outs = f(table, idxr)
    return tuple(jnp.transpose(o, (1, 0, 2)) for o in outs)


def kernel(history_context_features, emb_weight):
    idx_t = jnp.transpose(history_context_features, (2, 1, 0))
    idxr = jnp.transpose(idx_t.reshape(S, L, NW, C), (2, 0, 1, 3))
    return _run(emb_weight, idxr)
